# Initial kernel scaffold; baseline (speedup 1.0000x reference)
import jax
import jax.numpy as jnp
from jax.experimental import pallas as pl


def kernel(X, edge_index, unused, W1, b1, L1W, L1b, W2, b2, L2W, L2b, W3, b3, L3W, L3b, FCW, FCb):
    raise NotImplementedError("write your pallas kernel here")


# speedup vs baseline: 6.7029x; 6.7029x over previous
import functools

import jax
import jax.numpy as jnp
from jax import lax
from jax.experimental import pallas as pl
from jax.experimental.pallas import tpu as pltpu
from jax.experimental.pallas import tpu_sc as plsc

N = 10000
E = 320000
D = 128
D_OUT = 64

NC = 2
NS = 16
NW = NC * NS
NPAD = 10240
ROWS_PER_SUB = NPAD // NS
CHUNK = 128
NCHUNK = 80
EPT_PAD = CHUNK * NCHUNK
E_PAD = EPT_PAD * NW
DEGW = 128

BR = 1024


def _sc_mesh():
    return plsc.VectorSubcoreMesh(core_axis_name="c", subcore_axis_name="s")


def _sc_degree(dst3, ones, zdeg):
    @functools.partial(
        pl.kernel,
        out_type=jax.ShapeDtypeStruct((NC, NPAD, DEGW), jnp.float32),
        mesh=_sc_mesh(),
        scratch_types=[
            pltpu.VMEM((NCHUNK, CHUNK), jnp.int32),
            pltpu.VMEM((CHUNK, DEGW), jnp.float32),
            pltpu.VMEM_SHARED((NPAD, DEGW), jnp.float32),
        ],
    )
    def k(dst_hbm, ones_hbm, zdeg_hbm, out_hbm, dst_v, ones_v, deg_sh):
        c = lax.axis_index("c")
        s = lax.axis_index("s")
        w = c * NS + s
        slab = pl.ds(s * ROWS_PER_SUB, ROWS_PER_SUB)
        pltpu.sync_copy(zdeg_hbm.at[slab], deg_sh.at[slab])
        pltpu.sync_copy(ones_hbm, ones_v)
        pltpu.sync_copy(dst_hbm.at[w], dst_v)
        plsc.subcore_barrier()

        def body(j, carry):
            pltpu.sync_copy(ones_v, deg_sh.at[dst_v.at[j]], add=True)
            return carry

        lax.fori_loop(0, NCHUNK, body, 0)
        plsc.subcore_barrier()
        pltpu.sync_copy(deg_sh.at[slab], out_hbm.at[c, slab])

    return k(dst3, ones, zdeg)


def _sc_scatter(table, src3, dst3, zmain):
    @functools.partial(
        pl.kernel,
        out_type=jax.ShapeDtypeStruct((NC, NPAD, D), jnp.float32),
        mesh=_sc_mesh(),
        scratch_types=[
            pltpu.VMEM((NCHUNK, CHUNK), jnp.int32),
            pltpu.VMEM((NCHUNK, CHUNK), jnp.int32),
            pltpu.VMEM((CHUNK, D), jnp.float32),
            pltpu.VMEM_SHARED((NPAD, D), jnp.float32),
            pltpu.SemaphoreType.DMA,
        ],
    )
    def k(table_hbm, src_hbm, dst_hbm, z_hbm, out_hbm,
          src_v, dst_v, rows_v, acc_sh, sem):
        c = lax.axis_index("c")
        s = lax.axis_index("s")
        w = c * NS + s
        slab = pl.ds(s * ROWS_PER_SUB, ROWS_PER_SUB)
        pltpu.sync_copy(z_hbm.at[slab], acc_sh.at[slab])
        pltpu.sync_copy(src_hbm.at[w], src_v)
        pltpu.sync_copy(dst_hbm.at[w], dst_v)
        plsc.subcore_barrier()

        def body(j, carry):
            pltpu.async_copy(table_hbm.at[src_v.at[j]], rows_v, sem).wait()
            pltpu.sync_copy(rows_v, acc_sh.at[dst_v.at[j]], add=True)
            return carry

        lax.fori_loop(0, NCHUNK, body, 0)
        plsc.subcore_barrier()
        pltpu.sync_copy(acc_sh.at[slab], out_hbm.at[c, slab])

    return k(table, src3, dst3, zmain)


def _dinv_block(dg_ref, i):
    deg = dg_ref[0][:, 0:1] + dg_ref[1][:, 0:1] + 1.0
    rows = i * BR + lax.broadcasted_iota(jnp.int32, (BR, 1), 0)
    return jnp.where(rows < N, lax.rsqrt(deg), 0.0)


def _leaky(x):
    return jnp.where(x > 0, x, 0.01 * x)


_row_spec = pl.BlockSpec((BR, D), lambda i: (i, 0))
_deg_spec = pl.BlockSpec((NC, BR, DEGW), lambda i: (0, i, 0))
_s_spec = pl.BlockSpec((NC, BR, D), lambda i: (0, i, 0))
_w_spec = pl.BlockSpec((D, D), lambda i: (0, 0))
_b_spec = pl.BlockSpec((1, D), lambda i: (0, 0))


def _tc_first(x, w, lw, lb, degacc):
    def body(x_ref, w_ref, lw_ref, lb_ref, dg_ref, xws_ref, lin_ref):
        i = pl.program_id(0)
        dinv = _dinv_block(dg_ref, i)
        xv = x_ref[...]
        xw = jnp.dot(xv, w_ref[...], preferred_element_type=jnp.float32)
        xws_ref[...] = xw * dinv
        lin_ref[...] = (
            jnp.dot(xv, lw_ref[...], preferred_element_type=jnp.float32)
            + lb_ref[...]
        )

    return pl.pallas_call(
        body,
        grid=(NPAD // BR,),
        in_specs=[_row_spec, _w_spec, _w_spec, _b_spec, _deg_spec],
        out_specs=[_row_spec, _row_spec],
        out_shape=[
            jax.ShapeDtypeStruct((NPAD, D), jnp.float32),
            jax.ShapeDtypeStruct((NPAD, D), jnp.float32),
        ],
    )(x, w, lw, lb, degacc)


def _tc_mid(s2, xws, lin, b, degacc, w, lw, lb):
    def body(s_ref, xws_ref, lin_ref, b_ref, dg_ref, w_ref, lw_ref, lb_ref,
             xws_o, lin_o):
        i = pl.program_id(0)
        dinv = _dinv_block(dg_ref, i)
        h = (s_ref[0] + s_ref[1] + xws_ref[...]) * dinv + b_ref[...] + lin_ref[...]
        h = _leaky(h)
        xws_o[...] = jnp.dot(h, w_ref[...], preferred_element_type=jnp.float32) * dinv
        lin_o[...] = (
            jnp.dot(h, lw_ref[...], preferred_element_type=jnp.float32)
            + lb_ref[...]
        )

    return pl.pallas_call(
        body,
        grid=(NPAD // BR,),
        in_specs=[_s_spec, _row_spec, _row_spec, _b_spec, _deg_spec,
                  _w_spec, _w_spec, _b_spec],
        out_specs=[_row_spec, _row_spec],
        out_shape=[
            jax.ShapeDtypeStruct((NPAD, D), jnp.float32),
            jax.ShapeDtypeStruct((NPAD, D), jnp.float32),
        ],
    )(s2, xws, lin, b, degacc, w, lw, lb)


def _tc_final(s2, xws, lin, b, degacc, fcw, fcb):
    def body(s_ref, xws_ref, lin_ref, b_ref, dg_ref, fcw_ref, fcb_ref,
             h_o, y_o):
        i = pl.program_id(0)
        dinv = _dinv_block(dg_ref, i)
        h = (s_ref[0] + s_ref[1] + xws_ref[...]) * dinv + b_ref[...] + lin_ref[...]
        h = _leaky(h)
        h_o[...] = h
        y_o[...] = (
            jnp.dot(h, fcw_ref[...], preferred_element_type=jnp.float32)
            + fcb_ref[...]
        )

    return pl.pallas_call(
        body,
        grid=(NPAD // BR,),
        in_specs=[_s_spec, _row_spec, _row_spec, _b_spec, _deg_spec,
                  pl.BlockSpec((D, D_OUT), lambda i: (0, 0)),
                  pl.BlockSpec((1, D_OUT), lambda i: (0, 0))],
        out_specs=[_row_spec, pl.BlockSpec((BR, D_OUT), lambda i: (i, 0))],
        out_shape=[
            jax.ShapeDtypeStruct((NPAD, D), jnp.float32),
            jax.ShapeDtypeStruct((NPAD, D_OUT), jnp.float32),
        ],
    )(s2, xws, lin, b, degacc, fcw, fcb)


def kernel(X, edge_index, unused, W1, b1, L1W, L1b, W2, b2, L2W, L2b,
           W3, b3, L3W, L3b, FCW, FCb):
    pad_e = E_PAD - E
    src3 = jnp.concatenate(
        [edge_index[0], jnp.full((pad_e,), N, jnp.int32)]
    ).reshape(NW, NCHUNK, CHUNK)
    dst3 = jnp.concatenate(
        [edge_index[1], jnp.full((pad_e,), N, jnp.int32)]
    ).reshape(NW, NCHUNK, CHUNK)
    x_pad = jnp.concatenate(
        [X, jnp.zeros((NPAD - N, D), jnp.float32)], axis=0
    )
    ones = jnp.ones((CHUNK, DEGW), jnp.float32)
    zdeg = jnp.zeros((NPAD, DEGW), jnp.float32)
    zmain = jnp.zeros((NPAD, D), jnp.float32)
    b1r, b2r, b3r = b1.reshape(1, D), b2.reshape(1, D), b3.reshape(1, D)
    l1br, l2br, l3br = L1b.reshape(1, D), L2b.reshape(1, D), L3b.reshape(1, D)
    fcbr = FCb.reshape(1, D_OUT)

    degacc = _sc_degree(dst3, ones, zdeg)

    xws1, lin1 = _tc_first(x_pad, W1, L1W, l1br, degacc)
    s1 = _sc_scatter(xws1, src3, dst3, zmain)

    xws2, lin2 = _tc_mid(s1, xws1, lin1, b1r, degacc, W2, L2W, l2br)
    s2 = _sc_scatter(xws2, src3, dst3, zmain)

    xws3, lin3 = _tc_mid(s2, xws2, lin2, b2r, degacc, W3, L3W, l3br)
    s3 = _sc_scatter(xws3, src3, dst3, zmain)

    h_pad, y_pad = _tc_final(s3, xws3, lin3, b3r, degacc, FCW, fcbr)
    return (h_pad[:N], y_pad[:N])


# speedup vs baseline: 7.5943x; 1.1330x over previous
import functools

import jax
import jax.numpy as jnp
from jax import lax
from jax.experimental import pallas as pl
from jax.experimental.pallas import tpu as pltpu
from jax.experimental.pallas import tpu_sc as plsc

N = 10000
E = 320000
D = 128
D_OUT = 64

NC = 2
NS = 16
NW = NC * NS
NPAD = 10240
ROWS_PER_SUB = NPAD // NS
CHUNK = 128
NCHUNK = 80
EPT_PAD = CHUNK * NCHUNK
E_PAD = EPT_PAD * NW
DEGW = 128

BR = 1024


def _sc_mesh():
    return plsc.VectorSubcoreMesh(core_axis_name="c", subcore_axis_name="s")


def _sc_degree(dst3, ones, zdeg):
    @functools.partial(
        pl.kernel,
        out_type=jax.ShapeDtypeStruct((NC, NPAD, DEGW), jnp.float32),
        mesh=_sc_mesh(),
        scratch_types=[
            pltpu.VMEM((NCHUNK, CHUNK), jnp.int32),
            pltpu.VMEM((CHUNK, DEGW), jnp.float32),
            pltpu.VMEM_SHARED((NPAD, DEGW), jnp.float32),
            pltpu.SemaphoreType.DMA,
            pltpu.SemaphoreType.DMA,
        ],
    )
    def k(dst_hbm, ones_hbm, zdeg_hbm, out_hbm, dst_v, ones_v, deg_sh,
          sem0, sem1):
        sems = (sem0, sem1)
        c = lax.axis_index("c")
        s = lax.axis_index("s")
        w = c * NS + s
        slab = pl.ds(s * ROWS_PER_SUB, ROWS_PER_SUB)
        pltpu.sync_copy(zdeg_hbm.at[slab], deg_sh.at[slab])
        pltpu.sync_copy(ones_hbm, ones_v)
        pltpu.sync_copy(dst_hbm.at[w], dst_v)
        plsc.subcore_barrier()

        for b in range(2):
            pltpu.async_copy(ones_v, deg_sh.at[dst_v.at[b]], sems[b], add=True)

        @pl.loop(1, NCHUNK // 2)
        def _(i):
            for b in range(2):
                j = 2 * i + b
                pltpu.make_async_copy(
                    ones_v, deg_sh.at[dst_v.at[j - 2]], sems[b]
                ).wait()
                pltpu.async_copy(ones_v, deg_sh.at[dst_v.at[j]], sems[b],
                                 add=True)

        for b in range(2):
            pltpu.make_async_copy(
                ones_v, deg_sh.at[dst_v.at[NCHUNK - 2 + b]], sems[b]
            ).wait()
        plsc.subcore_barrier()
        pltpu.sync_copy(deg_sh.at[slab], out_hbm.at[c, slab])

    return k(dst3, ones, zdeg)


NBUF = 2
DRING = 4
_NSTEP = NCHUNK // DRING


def _sc_scatter(table, src3, dst3, zmain):
    @functools.partial(
        pl.kernel,
        out_type=jax.ShapeDtypeStruct((NC, NPAD, D), jnp.float32),
        mesh=_sc_mesh(),
        scratch_types=[
            pltpu.VMEM((NCHUNK, CHUNK), jnp.int32),
            [pltpu.VMEM((CHUNK,), jnp.int32) for _ in range(DRING)],
            pltpu.VMEM((NBUF, CHUNK, D), jnp.float32),
            pltpu.VMEM_SHARED((NPAD, D), jnp.float32),
            [pltpu.SemaphoreType.DMA for _ in range(NBUF)],
            [pltpu.SemaphoreType.DMA for _ in range(NBUF)],
            [pltpu.SemaphoreType.DMA for _ in range(DRING)],
        ],
    )
    def k(table_hbm, src_hbm, dst_hbm, z_hbm, out_hbm,
          src_v, dbuf, rows_v, acc_sh, gsem, ssem, dsem):
        c = lax.axis_index("c")
        s = lax.axis_index("s")
        w = c * NS + s
        slab = pl.ds(s * ROWS_PER_SUB, ROWS_PER_SUB)
        pltpu.sync_copy(z_hbm.at[slab], acc_sh.at[slab])
        pltpu.sync_copy(src_hbm.at[w], src_v)
        plsc.subcore_barrier()

        def gather(j, b):
            pltpu.async_copy(table_hbm.at[src_v.at[j]], rows_v.at[b], gsem[b])

        def dload(j, db):
            pltpu.async_copy(dst_hbm.at[w, j], dbuf[db], dsem[db])

        gather(0, 0)
        dload(0, 0)
        dload(1, 1)

        @pl.loop(0, _NSTEP)
        def _(i):
            for q in range(DRING):
                j = i * DRING + q
                b = q % NBUF
                b2 = (b + 1) % NBUF

                def wait_prev_scatter():
                    pltpu.make_async_copy(
                        rows_v.at[b2], acc_sh.at[dbuf[(q + 3) % DRING]],
                        ssem[b2],
                    ).wait()

                if q == 0:
                    pl.when(i >= 1)(wait_prev_scatter)
                else:
                    wait_prev_scatter()
                pl.when(j + 1 < NCHUNK)(lambda: gather(j + 1, b2))
                pl.when(j + 2 < NCHUNK)(
                    lambda: dload(j + 2, (q + 2) % DRING))
                pltpu.make_async_copy(
                    table_hbm.at[src_v.at[j]], rows_v.at[b], gsem[b]
                ).wait()
                pltpu.make_async_copy(
                    dst_hbm.at[w, j], dbuf[q], dsem[q]
                ).wait()
                pltpu.async_copy(rows_v.at[b], acc_sh.at[dbuf[q]],
                                 ssem[b], add=True)

        pltpu.make_async_copy(
            rows_v.at[(NCHUNK - 1) % NBUF],
            acc_sh.at[dbuf[(NCHUNK - 1) % DRING]],
            ssem[(NCHUNK - 1) % NBUF],
        ).wait()
        plsc.subcore_barrier()
        pltpu.sync_copy(acc_sh.at[slab], out_hbm.at[c, slab])

    return k(table, src3, dst3, zmain)


def _dinv_block(dg_ref, i):
    deg = dg_ref[0][:, 0:1] + dg_ref[1][:, 0:1] + 1.0
    rows = i * BR + lax.broadcasted_iota(jnp.int32, (BR, 1), 0)
    return jnp.where(rows < N, lax.rsqrt(deg), 0.0)


def _leaky(x):
    return jnp.where(x > 0, x, 0.01 * x)


_row_spec = pl.BlockSpec((BR, D), lambda i: (i, 0))
_deg_spec = pl.BlockSpec((NC, BR, DEGW), lambda i: (0, i, 0))
_s_spec = pl.BlockSpec((NC, BR, D), lambda i: (0, i, 0))
_w_spec = pl.BlockSpec((D, D), lambda i: (0, 0))
_b_spec = pl.BlockSpec((1, D), lambda i: (0, 0))


def _tc_first(x, w, lw, lb, degacc):
    def body(x_ref, w_ref, lw_ref, lb_ref, dg_ref, xws_ref, lin_ref):
        i = pl.program_id(0)
        dinv = _dinv_block(dg_ref, i)
        xv = x_ref[...]
        xw = jnp.dot(xv, w_ref[...], preferred_element_type=jnp.float32)
        xws_ref[...] = xw * dinv
        lin_ref[...] = (
            jnp.dot(xv, lw_ref[...], preferred_element_type=jnp.float32)
            + lb_ref[...]
        )

    return pl.pallas_call(
        body,
        grid=(NPAD // BR,),
        in_specs=[_row_spec, _w_spec, _w_spec, _b_spec, _deg_spec],
        out_specs=[_row_spec, _row_spec],
        out_shape=[
            jax.ShapeDtypeStruct((NPAD, D), jnp.float32),
            jax.ShapeDtypeStruct((NPAD, D), jnp.float32),
        ],
    )(x, w, lw, lb, degacc)


def _tc_mid(s2, xws, lin, b, degacc, w, lw, lb):
    def body(s_ref, xws_ref, lin_ref, b_ref, dg_ref, w_ref, lw_ref, lb_ref,
             xws_o, lin_o):
        i = pl.program_id(0)
        dinv = _dinv_block(dg_ref, i)
        h = (s_ref[0] + s_ref[1] + xws_ref[...]) * dinv + b_ref[...] + lin_ref[...]
        h = _leaky(h)
        xws_o[...] = jnp.dot(h, w_ref[...], preferred_element_type=jnp.float32) * dinv
        lin_o[...] = (
            jnp.dot(h, lw_ref[...], preferred_element_type=jnp.float32)
            + lb_ref[...]
        )

    return pl.pallas_call(
        body,
        grid=(NPAD // BR,),
        in_specs=[_s_spec, _row_spec, _row_spec, _b_spec, _deg_spec,
                  _w_spec, _w_spec, _b_spec],
        out_specs=[_row_spec, _row_spec],
        out_shape=[
            jax.ShapeDtypeStruct((NPAD, D), jnp.float32),
            jax.ShapeDtypeStruct((NPAD, D), jnp.float32),
        ],
    )(s2, xws, lin, b, degacc, w, lw, lb)


def _tc_final(s2, xws, lin, b, degacc, fcw, fcb):
    def body(s_ref, xws_ref, lin_ref, b_ref, dg_ref, fcw_ref, fcb_ref,
             h_o, y_o):
        i = pl.program_id(0)
        dinv = _dinv_block(dg_ref, i)
        h = (s_ref[0] + s_ref[1] + xws_ref[...]) * dinv + b_ref[...] + lin_ref[...]
        h = _leaky(h)
        h_o[...] = h
        y_o[...] = (
            jnp.dot(h, fcw_ref[...], preferred_element_type=jnp.float32)
            + fcb_ref[...]
        )

    return pl.pallas_call(
        body,
        grid=(NPAD // BR,),
        in_specs=[_s_spec, _row_spec, _row_spec, _b_spec, _deg_spec,
                  pl.BlockSpec((D, D_OUT), lambda i: (0, 0)),
                  pl.BlockSpec((1, D_OUT), lambda i: (0, 0))],
        out_specs=[_row_spec, pl.BlockSpec((BR, D_OUT), lambda i: (i, 0))],
        out_shape=[
            jax.ShapeDtypeStruct((NPAD, D), jnp.float32),
            jax.ShapeDtypeStruct((NPAD, D_OUT), jnp.float32),
        ],
    )(s2, xws, lin, b, degacc, fcw, fcb)


def kernel(X, edge_index, unused, W1, b1, L1W, L1b, W2, b2, L2W, L2b,
           W3, b3, L3W, L3b, FCW, FCb):
    pad_e = E_PAD - E
    src3 = jnp.concatenate(
        [edge_index[0], jnp.full((pad_e,), N, jnp.int32)]
    ).reshape(NW, NCHUNK, CHUNK)
    dst3 = jnp.concatenate(
        [edge_index[1], jnp.full((pad_e,), N, jnp.int32)]
    ).reshape(NW, NCHUNK, CHUNK)
    x_pad = jnp.concatenate(
        [X, jnp.zeros((NPAD - N, D), jnp.float32)], axis=0
    )
    ones = jnp.ones((CHUNK, DEGW), jnp.float32)
    zdeg = jnp.zeros((NPAD, DEGW), jnp.float32)
    zmain = jnp.zeros((NPAD, D), jnp.float32)
    b1r, b2r, b3r = b1.reshape(1, D), b2.reshape(1, D), b3.reshape(1, D)
    l1br, l2br, l3br = L1b.reshape(1, D), L2b.reshape(1, D), L3b.reshape(1, D)
    fcbr = FCb.reshape(1, D_OUT)

    degacc = _sc_degree(dst3, ones, zdeg)

    xws1, lin1 = _tc_first(x_pad, W1, L1W, l1br, degacc)
    s1 = _sc_scatter(xws1, src3, dst3, zmain)

    xws2, lin2 = _tc_mid(s1, xws1, lin1, b1r, degacc, W2, L2W, l2br)
    s2 = _sc_scatter(xws2, src3, dst3, zmain)

    xws3, lin3 = _tc_mid(s2, xws2, lin2, b2r, degacc, W3, L3W, l3br)
    s3 = _sc_scatter(xws3, src3, dst3, zmain)

    h_pad, y_pad = _tc_final(s3, xws3, lin3, b3r, degacc, FCW, fcbr)
    return (h_pad[:N], y_pad[:N])


# speedup vs baseline: 23.7139x; 3.1226x over previous
import functools

import jax
import jax.numpy as jnp
from jax import lax
from jax.experimental import pallas as pl
from jax.experimental.pallas import tpu as pltpu
from jax.experimental.pallas import tpu_sc as plsc

N = 10000
E = 320000
D = 128
D_OUT = 64

NC = 2
NS = 16
NW = NC * NS
NPAD = 10240
ROWS_PER_SUB = NPAD // NS
CHUNK = 128
NCHUNK = 80
EPT_PAD = CHUNK * NCHUNK
E_PAD = EPT_PAD * NW
DEGW = 128

BR = 1024


def _sc_mesh():
    return plsc.VectorSubcoreMesh(core_axis_name="c", subcore_axis_name="s")


def _sc_degree(dst3, ones, zdeg):
    @functools.partial(
        pl.kernel,
        out_type=jax.ShapeDtypeStruct((NC, NPAD, DEGW), jnp.float32),
        mesh=_sc_mesh(),
        scratch_types=[
            pltpu.VMEM((NCHUNK, CHUNK), jnp.int32),
            pltpu.VMEM((CHUNK, DEGW), jnp.float32),
            pltpu.VMEM_SHARED((NPAD, DEGW), jnp.float32),
            pltpu.SemaphoreType.DMA,
            pltpu.SemaphoreType.DMA,
        ],
    )
    def k(dst_hbm, ones_hbm, zdeg_hbm, out_hbm, dst_v, ones_v, deg_sh,
          sem0, sem1):
        sems = (sem0, sem1)
        c = lax.axis_index("c")
        s = lax.axis_index("s")
        w = c * NS + s
        slab = pl.ds(s * ROWS_PER_SUB, ROWS_PER_SUB)
        pltpu.sync_copy(zdeg_hbm.at[slab], deg_sh.at[slab])
        pltpu.sync_copy(ones_hbm, ones_v)
        pltpu.sync_copy(dst_hbm.at[w], dst_v)
        plsc.subcore_barrier()

        for b in range(2):
            pltpu.async_copy(ones_v, deg_sh.at[dst_v.at[b]], sems[b], add=True)

        @pl.loop(1, NCHUNK // 2)
        def _(i):
            for b in range(2):
                j = 2 * i + b
                pltpu.make_async_copy(
                    ones_v, deg_sh.at[dst_v.at[j - 2]], sems[b]
                ).wait()
                pltpu.async_copy(ones_v, deg_sh.at[dst_v.at[j]], sems[b],
                                 add=True)

        for b in range(2):
            pltpu.make_async_copy(
                ones_v, deg_sh.at[dst_v.at[NCHUNK - 2 + b]], sems[b]
            ).wait()
        plsc.subcore_barrier()
        pltpu.sync_copy(deg_sh.at[slab], out_hbm.at[c, slab])

    return k(dst3, ones, zdeg)


NBUF = 2
DRING = 4
_NSTEP = NCHUNK // DRING


def _sc_scatter(table, src3, dst3, zmain):
    @functools.partial(
        pl.kernel,
        out_type=jax.ShapeDtypeStruct((NC, NPAD, D), jnp.float32),
        mesh=_sc_mesh(),
        scratch_types=[
            pltpu.VMEM((NCHUNK, CHUNK), jnp.int32),
            [pltpu.VMEM((CHUNK,), jnp.int32) for _ in range(DRING)],
            pltpu.VMEM((NBUF, CHUNK, D), jnp.float32),
            pltpu.VMEM_SHARED((NPAD, D), jnp.float32),
            [pltpu.SemaphoreType.DMA for _ in range(NBUF)],
            [pltpu.SemaphoreType.DMA for _ in range(NBUF)],
            [pltpu.SemaphoreType.DMA for _ in range(DRING)],
        ],
    )
    def k(table_hbm, src_hbm, dst_hbm, z_hbm, out_hbm,
          src_v, dbuf, rows_v, acc_sh, gsem, ssem, dsem):
        c = lax.axis_index("c")
        s = lax.axis_index("s")
        w = c * NS + s
        slab = pl.ds(s * ROWS_PER_SUB, ROWS_PER_SUB)
        pltpu.sync_copy(z_hbm.at[slab], acc_sh.at[slab])
        pltpu.sync_copy(src_hbm.at[w], src_v)
        plsc.subcore_barrier()

        def gather(j, b):
            pltpu.async_copy(table_hbm.at[src_v.at[j]], rows_v.at[b], gsem[b])

        def dload(j, db):
            pltpu.async_copy(dst_hbm.at[w, j], dbuf[db], dsem[db])

        gather(0, 0)
        dload(0, 0)
        dload(1, 1)

        @pl.loop(0, _NSTEP)
        def _(i):
            for q in range(DRING):
                j = i * DRING + q
                b = q % NBUF
                b2 = (b + 1) % NBUF

                def wait_prev_scatter():
                    pltpu.make_async_copy(
                        rows_v.at[b2], acc_sh.at[dbuf[(q + 3) % DRING]],
                        ssem[b2],
                    ).wait()

                if q == 0:
                    pl.when(i >= 1)(wait_prev_scatter)
                else:
                    wait_prev_scatter()
                pl.when(j + 1 < NCHUNK)(lambda: gather(j + 1, b2))
                pl.when(j + 2 < NCHUNK)(
                    lambda: dload(j + 2, (q + 2) % DRING))
                pltpu.make_async_copy(
                    table_hbm.at[src_v.at[j]], rows_v.at[b], gsem[b]
                ).wait()
                pltpu.make_async_copy(
                    dst_hbm.at[w, j], dbuf[q], dsem[q]
                ).wait()
                pltpu.async_copy(rows_v.at[b], acc_sh.at[dbuf[q]],
                                 ssem[b], add=True)

        pltpu.make_async_copy(
            rows_v.at[(NCHUNK - 1) % NBUF],
            acc_sh.at[dbuf[(NCHUNK - 1) % DRING]],
            ssem[(NCHUNK - 1) % NBUF],
        ).wait()
        plsc.subcore_barrier()
        pltpu.sync_copy(acc_sh.at[slab], out_hbm.at[c, slab])

    return k(table, src3, dst3, zmain)


def _dinv_block(dg_ref, i):
    deg = dg_ref[0][:, 0:1] + dg_ref[1][:, 0:1] + 1.0
    rows = i * BR + lax.broadcasted_iota(jnp.int32, (BR, 1), 0)
    return jnp.where(rows < N, lax.rsqrt(deg), 0.0)


def _leaky(x):
    return jnp.where(x > 0, x, 0.01 * x)


_row_spec = pl.BlockSpec((BR, D), lambda i: (i, 0))
_deg_spec = pl.BlockSpec((NC, BR, DEGW), lambda i: (0, i, 0))
_s_spec = pl.BlockSpec((NC, BR, D), lambda i: (0, i, 0))
_w_spec = pl.BlockSpec((D, D), lambda i: (0, 0))
_b_spec = pl.BlockSpec((1, D), lambda i: (0, 0))


def _tc_first(x, w, lw, lb, degacc):
    def body(x_ref, w_ref, lw_ref, lb_ref, dg_ref, xws_ref, lin_ref):
        i = pl.program_id(0)
        dinv = _dinv_block(dg_ref, i)
        xv = x_ref[...]
        xw = jnp.dot(xv, w_ref[...], preferred_element_type=jnp.float32)
        xws_ref[...] = xw * dinv
        lin_ref[...] = (
            jnp.dot(xv, lw_ref[...], preferred_element_type=jnp.float32)
            + lb_ref[...]
        )

    return pl.pallas_call(
        body,
        grid=(NPAD // BR,),
        in_specs=[_row_spec, _w_spec, _w_spec, _b_spec, _deg_spec],
        out_specs=[_row_spec, _row_spec],
        out_shape=[
            jax.ShapeDtypeStruct((NPAD, D), jnp.float32),
            jax.ShapeDtypeStruct((NPAD, D), jnp.float32),
        ],
    )(x, w, lw, lb, degacc)


def _tc_mid(s2, xws, lin, b, degacc, w, lw, lb):
    def body(s_ref, xws_ref, lin_ref, b_ref, dg_ref, w_ref, lw_ref, lb_ref,
             xws_o, lin_o):
        i = pl.program_id(0)
        dinv = _dinv_block(dg_ref, i)
        h = (s_ref[0] + s_ref[1] + xws_ref[...]) * dinv + b_ref[...] + lin_ref[...]
        h = _leaky(h)
        xws_o[...] = jnp.dot(h, w_ref[...], preferred_element_type=jnp.float32) * dinv
        lin_o[...] = (
            jnp.dot(h, lw_ref[...], preferred_element_type=jnp.float32)
            + lb_ref[...]
        )

    return pl.pallas_call(
        body,
        grid=(NPAD // BR,),
        in_specs=[_s_spec, _row_spec, _row_spec, _b_spec, _deg_spec,
                  _w_spec, _w_spec, _b_spec],
        out_specs=[_row_spec, _row_spec],
        out_shape=[
            jax.ShapeDtypeStruct((NPAD, D), jnp.float32),
            jax.ShapeDtypeStruct((NPAD, D), jnp.float32),
        ],
    )(s2, xws, lin, b, degacc, w, lw, lb)


def _tc_final(s2, xws, lin, b, degacc, fcw, fcb):
    def body(s_ref, xws_ref, lin_ref, b_ref, dg_ref, fcw_ref, fcb_ref,
             h_o, y_o):
        i = pl.program_id(0)
        dinv = _dinv_block(dg_ref, i)
        h = (s_ref[0] + s_ref[1] + xws_ref[...]) * dinv + b_ref[...] + lin_ref[...]
        h = _leaky(h)
        h_o[...] = h
        y_o[...] = (
            jnp.dot(h, fcw_ref[...], preferred_element_type=jnp.float32)
            + fcb_ref[...]
        )

    return pl.pallas_call(
        body,
        grid=(NPAD // BR,),
        in_specs=[_s_spec, _row_spec, _row_spec, _b_spec, _deg_spec,
                  pl.BlockSpec((D, D_OUT), lambda i: (0, 0)),
                  pl.BlockSpec((1, D_OUT), lambda i: (0, 0))],
        out_specs=[_row_spec, pl.BlockSpec((BR, D_OUT), lambda i: (i, 0))],
        out_shape=[
            jax.ShapeDtypeStruct((NPAD, D), jnp.float32),
            jax.ShapeDtypeStruct((NPAD, D_OUT), jnp.float32),
        ],
    )(s2, xws, lin, b, degacc, fcw, fcb)


def kernel(X, edge_index, unused, W1, b1, L1W, L1b, W2, b2, L2W, L2b,
           W3, b3, L3W, L3b, FCW, FCb):
    pad_e = E_PAD - E
    pad_idx = N + jnp.arange(pad_e, dtype=jnp.int32) % (NPAD - N)
    src3 = jnp.concatenate([edge_index[0], pad_idx]).reshape(NW, NCHUNK, CHUNK)
    dst3 = jnp.concatenate([edge_index[1], pad_idx]).reshape(NW, NCHUNK, CHUNK)
    x_pad = jnp.concatenate(
        [X, jnp.zeros((NPAD - N, D), jnp.float32)], axis=0
    )
    ones = jnp.ones((CHUNK, DEGW), jnp.float32)
    zdeg = jnp.zeros((NPAD, DEGW), jnp.float32)
    zmain = jnp.zeros((NPAD, D), jnp.float32)
    b1r, b2r, b3r = b1.reshape(1, D), b2.reshape(1, D), b3.reshape(1, D)
    l1br, l2br, l3br = L1b.reshape(1, D), L2b.reshape(1, D), L3b.reshape(1, D)
    fcbr = FCb.reshape(1, D_OUT)

    degacc = _sc_degree(dst3, ones, zdeg)

    xws1, lin1 = _tc_first(x_pad, W1, L1W, l1br, degacc)
    s1 = _sc_scatter(xws1, src3, dst3, zmain)

    xws2, lin2 = _tc_mid(s1, xws1, lin1, b1r, degacc, W2, L2W, l2br)
    s2 = _sc_scatter(xws2, src3, dst3, zmain)

    xws3, lin3 = _tc_mid(s2, xws2, lin2, b2r, degacc, W3, L3W, l3br)
    s3 = _sc_scatter(xws3, src3, dst3, zmain)

    h_pad, y_pad = _tc_final(s3, xws3, lin3, b3r, degacc, FCW, fcbr)
    return (h_pad[:N], y_pad[:N])


# speedup vs baseline: 23.8469x; 1.0056x over previous
import functools

import jax
import jax.numpy as jnp
from jax import lax
from jax.experimental import pallas as pl
from jax.experimental.pallas import tpu as pltpu
from jax.experimental.pallas import tpu_sc as plsc

N = 10000
E = 320000
D = 128
D_OUT = 64

NC = 2
NS = 16
NW = NC * NS
NPAD = 10240
ROWS_PER_SUB = NPAD // NS
CHUNK = 128
NCHUNK = 80
EPT_PAD = CHUNK * NCHUNK
E_PAD = EPT_PAD * NW
DEGW = 128

BR = 1024


def _sc_mesh():
    return plsc.VectorSubcoreMesh(core_axis_name="c", subcore_axis_name="s")


def _sc_degree(dst3, ones, zdeg):
    @functools.partial(
        pl.kernel,
        out_type=jax.ShapeDtypeStruct((NC, NPAD, DEGW), jnp.float32),
        mesh=_sc_mesh(),
        scratch_types=[
            pltpu.VMEM((NCHUNK, CHUNK), jnp.int32),
            pltpu.VMEM((CHUNK, DEGW), jnp.float32),
            pltpu.VMEM_SHARED((NPAD, DEGW), jnp.float32),
            pltpu.SemaphoreType.DMA,
            pltpu.SemaphoreType.DMA,
        ],
    )
    def k(dst_hbm, ones_hbm, zdeg_hbm, out_hbm, dst_v, ones_v, deg_sh,
          sem0, sem1):
        sems = (sem0, sem1)
        c = lax.axis_index("c")
        s = lax.axis_index("s")
        w = c * NS + s
        slab = pl.ds(s * ROWS_PER_SUB, ROWS_PER_SUB)
        pltpu.sync_copy(zdeg_hbm.at[slab], deg_sh.at[slab])
        pltpu.sync_copy(ones_hbm, ones_v)
        pltpu.sync_copy(dst_hbm.at[w], dst_v)
        plsc.subcore_barrier()

        for b in range(2):
            pltpu.async_copy(ones_v, deg_sh.at[dst_v.at[b]], sems[b], add=True)

        @pl.loop(1, NCHUNK // 2)
        def _(i):
            for b in range(2):
                j = 2 * i + b
                pltpu.make_async_copy(
                    ones_v, deg_sh.at[dst_v.at[j - 2]], sems[b]
                ).wait()
                pltpu.async_copy(ones_v, deg_sh.at[dst_v.at[j]], sems[b],
                                 add=True)

        for b in range(2):
            pltpu.make_async_copy(
                ones_v, deg_sh.at[dst_v.at[NCHUNK - 2 + b]], sems[b]
            ).wait()
        plsc.subcore_barrier()
        pltpu.sync_copy(deg_sh.at[slab], out_hbm.at[c, slab])

    return k(dst3, ones, zdeg)


NBUF = 2
DRING = 4
_NSTEP = NCHUNK // DRING


def _sc_scatter(table, src3, dst3, zmain):
    @functools.partial(
        pl.kernel,
        out_type=jax.ShapeDtypeStruct((NC, NPAD, D), jnp.float32),
        mesh=_sc_mesh(),
        scratch_types=[
            pltpu.VMEM((NCHUNK, CHUNK), jnp.int32),
            [pltpu.VMEM((CHUNK,), jnp.int32) for _ in range(DRING)],
            pltpu.VMEM((NBUF, CHUNK, D), jnp.float32),
            pltpu.VMEM_SHARED((NPAD, D), jnp.float32),
            [pltpu.SemaphoreType.DMA for _ in range(NBUF)],
            [pltpu.SemaphoreType.DMA for _ in range(NBUF)],
            [pltpu.SemaphoreType.DMA for _ in range(DRING)],
        ],
    )
    def k(table_hbm, src_hbm, dst_hbm, z_hbm, out_hbm,
          src_v, dbuf, rows_v, acc_sh, gsem, ssem, dsem):
        c = lax.axis_index("c")
        s = lax.axis_index("s")
        w = c * NS + s
        slab = pl.ds(s * ROWS_PER_SUB, ROWS_PER_SUB)
        pltpu.sync_copy(z_hbm.at[slab], acc_sh.at[slab])
        pltpu.sync_copy(src_hbm.at[w], src_v)
        plsc.subcore_barrier()

        def gather(j, b):
            pltpu.async_copy(table_hbm.at[src_v.at[j]], rows_v.at[b], gsem[b])

        def dload(j, db):
            pltpu.async_copy(dst_hbm.at[w, j], dbuf[db], dsem[db])

        gather(0, 0)
        dload(0, 0)
        dload(1, 1)

        @pl.loop(0, _NSTEP)
        def _(i):
            for q in range(DRING):
                j = i * DRING + q
                b = q % NBUF
                b2 = (b + 1) % NBUF

                def wait_prev_scatter():
                    pltpu.make_async_copy(
                        rows_v.at[b2], acc_sh.at[dbuf[(q + 3) % DRING]],
                        ssem[b2],
                    ).wait()

                if q == 0:
                    pl.when(i >= 1)(wait_prev_scatter)
                else:
                    wait_prev_scatter()
                pl.when(j + 1 < NCHUNK)(lambda: gather(j + 1, b2))
                pl.when(j + 2 < NCHUNK)(
                    lambda: dload(j + 2, (q + 2) % DRING))
                pltpu.make_async_copy(
                    table_hbm.at[src_v.at[j]], rows_v.at[b], gsem[b]
                ).wait()
                pltpu.make_async_copy(
                    dst_hbm.at[w, j], dbuf[q], dsem[q]
                ).wait()
                pltpu.async_copy(rows_v.at[b], acc_sh.at[dbuf[q]],
                                 ssem[b], add=True)

        pltpu.make_async_copy(
            rows_v.at[(NCHUNK - 1) % NBUF],
            acc_sh.at[dbuf[(NCHUNK - 1) % DRING]],
            ssem[(NCHUNK - 1) % NBUF],
        ).wait()
        plsc.subcore_barrier()
        pltpu.sync_copy(acc_sh.at[slab], out_hbm.at[c, slab])

    return k(table, src3, dst3, zmain)


def _dinv_from_deg(dg_ref, i):
    deg = dg_ref[0][:, 0:1] + dg_ref[1][:, 0:1] + 1.0
    rows = i * BR + lax.broadcasted_iota(jnp.int32, (BR, 1), 0)
    return jnp.where(rows < N, lax.rsqrt(deg), 0.0)


def _leaky(x):
    return jnp.where(x > 0, x, 0.01 * x)


_row_spec = pl.BlockSpec((BR, D), lambda i: (i, 0))
_degacc_spec = pl.BlockSpec((NC, BR, DEGW), lambda i: (0, i, 0))
_dinv_spec = pl.BlockSpec((BR, 1), lambda i: (i, 0))
_s_spec = pl.BlockSpec((NC, BR, D), lambda i: (0, i, 0))
_w_spec = pl.BlockSpec((D, D), lambda i: (0, 0))
_b_spec = pl.BlockSpec((1, D), lambda i: (0, 0))


def _tc_first(x, w, lw, lb, degacc):
    def body(x_ref, w_ref, lw_ref, lb_ref, dg_ref, xws_ref, lin_ref, dinv_ref):
        i = pl.program_id(0)
        dinv = _dinv_from_deg(dg_ref, i)
        dinv_ref[...] = dinv
        xv = x_ref[...]
        xw = jnp.dot(xv, w_ref[...], preferred_element_type=jnp.float32)
        xws_ref[...] = xw * dinv
        lin_ref[...] = (
            jnp.dot(xv, lw_ref[...], preferred_element_type=jnp.float32)
            + lb_ref[...]
        )

    return pl.pallas_call(
        body,
        grid=(NPAD // BR,),
        in_specs=[_row_spec, _w_spec, _w_spec, _b_spec, _degacc_spec],
        out_specs=[_row_spec, _row_spec, _dinv_spec],
        out_shape=[
            jax.ShapeDtypeStruct((NPAD, D), jnp.float32),
            jax.ShapeDtypeStruct((NPAD, D), jnp.float32),
            jax.ShapeDtypeStruct((NPAD, 1), jnp.float32),
        ],
    )(x, w, lw, lb, degacc)


def _tc_mid(s2, xws, lin, b, degacc, w, lw, lb):
    def body(s_ref, xws_ref, lin_ref, b_ref, dg_ref, w_ref, lw_ref, lb_ref,
             xws_o, lin_o):
        dinv = dg_ref[...]
        h = (s_ref[0] + s_ref[1] + xws_ref[...]) * dinv + b_ref[...] + lin_ref[...]
        h = _leaky(h)
        xws_o[...] = jnp.dot(h, w_ref[...], preferred_element_type=jnp.float32) * dinv
        lin_o[...] = (
            jnp.dot(h, lw_ref[...], preferred_element_type=jnp.float32)
            + lb_ref[...]
        )

    return pl.pallas_call(
        body,
        grid=(NPAD // BR,),
        in_specs=[_s_spec, _row_spec, _row_spec, _b_spec, _dinv_spec,
                  _w_spec, _w_spec, _b_spec],
        out_specs=[_row_spec, _row_spec],
        out_shape=[
            jax.ShapeDtypeStruct((NPAD, D), jnp.float32),
            jax.ShapeDtypeStruct((NPAD, D), jnp.float32),
        ],
    )(s2, xws, lin, b, degacc, w, lw, lb)


def _tc_final(s2, xws, lin, b, degacc, fcw, fcb):
    def body(s_ref, xws_ref, lin_ref, b_ref, dg_ref, fcw_ref, fcb_ref,
             h_o, y_o):
        dinv = dg_ref[...]
        h = (s_ref[0] + s_ref[1] + xws_ref[...]) * dinv + b_ref[...] + lin_ref[...]
        h = _leaky(h)
        h_o[...] = h
        y_o[...] = (
            jnp.dot(h, fcw_ref[...], preferred_element_type=jnp.float32)
            + fcb_ref[...]
        )

    return pl.pallas_call(
        body,
        grid=(NPAD // BR,),
        in_specs=[_s_spec, _row_spec, _row_spec, _b_spec, _dinv_spec,
                  pl.BlockSpec((D, D_OUT), lambda i: (0, 0)),
                  pl.BlockSpec((1, D_OUT), lambda i: (0, 0))],
        out_specs=[_row_spec, pl.BlockSpec((BR, D_OUT), lambda i: (i, 0))],
        out_shape=[
            jax.ShapeDtypeStruct((NPAD, D), jnp.float32),
            jax.ShapeDtypeStruct((NPAD, D_OUT), jnp.float32),
        ],
    )(s2, xws, lin, b, degacc, fcw, fcb)


def kernel(X, edge_index, unused, W1, b1, L1W, L1b, W2, b2, L2W, L2b,
           W3, b3, L3W, L3b, FCW, FCb):
    pad_e = E_PAD - E
    pad_idx = N + jnp.arange(pad_e, dtype=jnp.int32) % (NPAD - N)
    src3 = jnp.concatenate([edge_index[0], pad_idx]).reshape(NW, NCHUNK, CHUNK)
    dst3 = jnp.concatenate([edge_index[1], pad_idx]).reshape(NW, NCHUNK, CHUNK)
    x_pad = jnp.concatenate(
        [X, jnp.zeros((NPAD - N, D), jnp.float32)], axis=0
    )

    zmain = jnp.zeros((NPAD, D), jnp.float32)
    b1r, b2r, b3r = b1.reshape(1, D), b2.reshape(1, D), b3.reshape(1, D)
    l1br, l2br, l3br = L1b.reshape(1, D), L2b.reshape(1, D), L3b.reshape(1, D)
    fcbr = FCb.reshape(1, D_OUT)

    ones = jnp.ones((CHUNK, DEGW), jnp.float32)
    zdeg = jnp.zeros((NPAD, DEGW), jnp.float32)
    degacc = _sc_degree(dst3, ones, zdeg)

    xws1, lin1, dinv = _tc_first(x_pad, W1, L1W, l1br, degacc)
    s1 = _sc_scatter(xws1, src3, dst3, zmain)

    xws2, lin2 = _tc_mid(s1, xws1, lin1, b1r, dinv, W2, L2W, l2br)
    s2 = _sc_scatter(xws2, src3, dst3, zmain)

    xws3, lin3 = _tc_mid(s2, xws2, lin2, b2r, dinv, W3, L3W, l3br)
    s3 = _sc_scatter(xws3, src3, dst3, zmain)

    h_pad, y_pad = _tc_final(s3, xws3, lin3, b3r, dinv, FCW, fcbr)
    return (h_pad[:N], y_pad[:N])
